# baseline (speedup 1.0000x reference)
import math

import jax
import jax.numpy as jnp
from jax import lax
from jax.experimental import pallas as pl
from jax.experimental.pallas import tpu as pltpu
from jax.experimental.pallas import tpu_sc as plsc

N_TOK = 4096
N_CLS = 32000
PAD = 0
SMOOTHING = 0.1
CONF = 1.0 - SMOOTHING
EPS = SMOOTHING / (N_CLS - 2)
C0 = (N_CLS - 2) * EPS * math.log(EPS) + CONF * math.log(CONF)

TC_ROWS = 2048
SC_ROWS = N_TOK - TC_ROWS

ROW_BLK = 128


def _tc_body(tgt_ref, x_ref, acc_ref):
    @pl.when(pl.program_id(0) == 0)
    def _():
        for q in range(4):
            acc_ref[0, q] = 0.0

    tgt = tgt_ref[...]
    m = tgt != PAD
    mf = m.astype(jnp.float32)
    xb = x_ref[...]
    col = lax.broadcasted_iota(jnp.int32, (ROW_BLK, N_CLS), 1)
    tgtm = jnp.where(m, tgt, -1)
    acc_ref[0, 0] += jnp.sum(xb * mf)
    acc_ref[0, 1] += jnp.sum(jnp.where(col == tgtm, xb, 0.0))
    acc_ref[0, 2] += jnp.sum(xb[:, 0:1] * mf)
    acc_ref[0, 3] += jnp.sum(mf)


_tc_pass = pl.pallas_call(
    _tc_body,
    grid=(TC_ROWS // ROW_BLK,),
    in_specs=[
        pl.BlockSpec((ROW_BLK, 1), lambda i: (i, 0)),
        pl.BlockSpec((ROW_BLK, N_CLS), lambda i: (i, 0)),
    ],
    out_specs=pl.BlockSpec((1, 4), lambda i: (0, 0), memory_space=pltpu.SMEM),
    out_shape=jax.ShapeDtypeStruct((1, 4), jnp.float32),
)

L = 16
NC, NS = 2, 16
NW = NC * NS
R_W = SC_ROWS // NW
N_GRP = R_W // 16
CW = 3200
N_CHK = N_CLS // CW


def _sc_body(x_hbm, tgt_hbm, out_hbm, dtgt_v, buf0, buf1, res_v,
             sem0, sem1):
    wid = lax.axis_index("s") * NC + lax.axis_index("c")
    lane_ids = lax.iota(jnp.int32, L)
    row0 = TC_ROWS + wid * R_W
    pltpu.sync_copy(tgt_hbm.at[pl.ds(row0, R_W)], dtgt_v)

    bufs = (buf0, buf1)
    sems = (sem0, sem1)
    zv = jnp.zeros((L,), jnp.float32)
    lane0 = jnp.where(lane_ids == 0, 1.0, 0.0).astype(jnp.float32)

    def _grp(g, carry):
        accd, accg, accb, acck = carry
        r0 = row0 + g * 16
        tv = dtgt_v[pl.ds(pl.multiple_of(g * 16, 16), 16)]
        t = [tv[r] for r in range(16)]
        mf = [jnp.full((L,), jnp.where(t[r] != PAD, 1.0, 0.0), jnp.float32)
              for r in range(16)]
        for r in range(16):
            acck = acck + lane0 * mf[r]

        cps = [None, None]
        cps[0] = pltpu.async_copy(
            x_hbm.at[pl.ds(r0, 16), pl.ds(0, CW)], bufs[0], sems[0])
        for c in range(N_CHK):
            s = c & 1
            if c + 1 < N_CHK:
                cps[(c + 1) & 1] = pltpu.async_copy(
                    x_hbm.at[pl.ds(r0, 16), pl.ds((c + 1) * CW, CW)],
                    bufs[(c + 1) & 1], sems[(c + 1) & 1])
            cps[s].wait()
            buf = bufs[s]

            def _chunk(k, ad):
                colb = pl.multiple_of(k * L, L)
                for r in range(16):
                    ad = ad + buf[r, pl.ds(colb, L)] * mf[r]
                return ad

            accd = lax.fori_loop(0, CW // L, _chunk, accd)

            for r in range(16):
                off = t[r] - c * CW
                safe = (off >= 0) & (off < CW)
                offc = jnp.where(safe, off, 0)
                lanebit = lax.rem(offc, L)
                start = pl.multiple_of(offc - lanebit, L)
                v = buf[r, pl.ds(start, L)]
                w = jnp.full((L,), jnp.where(safe, 1.0, 0.0), jnp.float32) * mf[r]
                lanev = jnp.full((L,), lanebit, jnp.int32)
                accg = accg + v * jnp.where(lane_ids == lanev, w, zv)
            if c == 0:
                for r in range(16):
                    accb = accb + buf[r, pl.ds(0, L)] * (lane0 * mf[r])
        return accd, accg, accb, acck

    zero = jnp.zeros((L,), jnp.float32)
    accd, accg, accb, acck = lax.fori_loop(
        0, N_GRP, _grp, (zero, zero, zero, zero))

    res_v[0, :] = accd
    res_v[1, :] = accg
    res_v[2, :] = accb
    res_v[3, :] = acck
    pltpu.sync_copy(res_v, out_hbm.at[wid])


_sc_pass = pl.kernel(
    _sc_body,
    out_type=jax.ShapeDtypeStruct((NW, 4, L), jnp.float32),
    mesh=plsc.VectorSubcoreMesh(core_axis_name="c", subcore_axis_name="s"),
    scratch_types=[
        pltpu.VMEM((R_W,), jnp.int32),
        pltpu.VMEM((16, CW), jnp.float32),
        pltpu.VMEM((16, CW), jnp.float32),
        pltpu.VMEM((4, L), jnp.float32),
        pltpu.SemaphoreType.DMA,
        pltpu.SemaphoreType.DMA,
    ],
)


def kernel(x, target):
    tgt = target.astype(jnp.int32)
    tc = _tc_pass(tgt.reshape(N_TOK, 1), x)
    res = _sc_pass(x, tgt)
    a = tc[0, 0] + jnp.sum(res[:, 0, :])
    g = tc[0, 1] + jnp.sum(res[:, 1, :])
    b = tc[0, 2] + jnp.sum(res[:, 2, :])
    k = tc[0, 3] + jnp.sum(res[:, 3, :])
    return k * C0 - EPS * a + EPS * b + (EPS - CONF) * g


# speedup vs baseline: 1.1203x; 1.1203x over previous
import math

import jax
import jax.numpy as jnp
from jax import lax
from jax.experimental import pallas as pl
from jax.experimental.pallas import tpu as pltpu
from jax.experimental.pallas import tpu_sc as plsc

N_TOK = 4096
N_CLS = 32000
PAD = 0
SMOOTHING = 0.1
CONF = 1.0 - SMOOTHING
EPS = SMOOTHING / (N_CLS - 2)
C0 = (N_CLS - 2) * EPS * math.log(EPS) + CONF * math.log(CONF)

TC_ROWS = 2304
SC_ROWS = N_TOK - TC_ROWS

ROW_BLK = 128


def _tc_body(tgt_ref, x_ref, acc_ref):
    @pl.when(pl.program_id(0) == 0)
    def _():
        for q in range(4):
            acc_ref[0, q] = 0.0

    tgt = tgt_ref[...]
    m = tgt != PAD
    mf = m.astype(jnp.float32)
    xb = x_ref[...]
    col = lax.broadcasted_iota(jnp.int32, (ROW_BLK, N_CLS), 1)
    tgtm = jnp.where(m, tgt, -1)
    acc_ref[0, 0] += jnp.sum(xb * mf)
    acc_ref[0, 1] += jnp.sum(jnp.where(col == tgtm, xb, 0.0))
    acc_ref[0, 2] += jnp.sum(xb[:, 0:1] * mf)
    acc_ref[0, 3] += jnp.sum(mf)


_tc_pass = pl.pallas_call(
    _tc_body,
    grid=(TC_ROWS // ROW_BLK,),
    in_specs=[
        pl.BlockSpec((ROW_BLK, 1), lambda i: (i, 0)),
        pl.BlockSpec((ROW_BLK, N_CLS), lambda i: (i, 0)),
    ],
    out_specs=pl.BlockSpec((1, 4), lambda i: (0, 0), memory_space=pltpu.SMEM),
    out_shape=jax.ShapeDtypeStruct((1, 4), jnp.float32),
)

L = 16
NC, NS = 2, 16
NW = NC * NS
R_W = SC_ROWS // NW
GR = 8
N_GRP = R_W // GR
CW = 6400
N_CHK = N_CLS // CW
N_TOT = N_GRP * N_CHK


def _sc_body(x_hbm, tgt_hbm, out_hbm, dtgt_v, buf0, buf1, res_v,
             sem0, sem1):
    wid = lax.axis_index("s") * NC + lax.axis_index("c")
    lane_ids = lax.iota(jnp.int32, L)
    row0 = TC_ROWS + wid * R_W
    pltpu.sync_copy(tgt_hbm.at[pl.ds(row0, R_W)], dtgt_v.at[pl.ds(0, R_W)])

    bufs = (buf0, buf1)
    sems = (sem0, sem1)
    zv = jnp.zeros((L,), jnp.float32)
    lane0 = jnp.where(lane_ids == 0, 1.0, 0.0).astype(jnp.float32)

    def _dma(n, s):
        g, c = divmod(n, N_CHK)
        return pltpu.async_copy(
            x_hbm.at[pl.ds(row0 + g * GR, GR), pl.ds(c * CW, CW)],
            bufs[s], sems[s])

    t, mf = [], []
    for g in range(N_GRP):
        tv = dtgt_v[pl.ds(g * GR, L)]
        t.append([tv[r] for r in range(GR)])
        mf.append([jnp.full((L,), jnp.where(tv[r] != PAD, 1.0, 0.0),
                            jnp.float32) for r in range(GR)])

    accd = zv
    accg = zv
    accb = zv
    acck = zv
    cps = [_dma(0, 0), _dma(1, 1)]
    for n in range(N_TOT):
        s = n & 1
        g, c = divmod(n, N_CHK)
        cps[s].wait()
        buf = bufs[s]

        def _chunk(k, ad):
            colb = pl.multiple_of(k * L, L)
            for r in range(GR):
                ad = ad + buf[r, pl.ds(colb, L)] * mf[g][r]
            return ad

        accd = lax.fori_loop(0, CW // L, _chunk, accd)

        for r in range(GR):
            off = t[g][r] - c * CW
            safe = (off >= 0) & (off < CW)
            offc = jnp.where(safe, off, 0)
            lanebit = lax.rem(offc, L)
            start = pl.multiple_of(offc - lanebit, L)
            v = buf[r, pl.ds(start, L)]
            w = jnp.full((L,), jnp.where(safe, 1.0, 0.0), jnp.float32) * mf[g][r]
            lanev = jnp.full((L,), lanebit, jnp.int32)
            accg = accg + v * jnp.where(lane_ids == lanev, w, zv)
        if c == 0:
            for r in range(GR):
                accb = accb + buf[r, pl.ds(0, L)] * (lane0 * mf[g][r])
                acck = acck + lane0 * mf[g][r]
        if n + 2 < N_TOT:
            cps[s] = _dma(n + 2, s)

    res_v[0, :] = accd
    res_v[1, :] = accg
    res_v[2, :] = accb
    res_v[3, :] = acck
    pltpu.sync_copy(res_v, out_hbm.at[wid])


_sc_pass = pl.kernel(
    _sc_body,
    out_type=jax.ShapeDtypeStruct((NW, 4, L), jnp.float32),
    mesh=plsc.VectorSubcoreMesh(core_axis_name="c", subcore_axis_name="s"),
    scratch_types=[
        pltpu.VMEM((R_W + L,), jnp.int32),
        pltpu.VMEM((GR, CW), jnp.float32),
        pltpu.VMEM((GR, CW), jnp.float32),
        pltpu.VMEM((4, L), jnp.float32),
        pltpu.SemaphoreType.DMA,
        pltpu.SemaphoreType.DMA,
    ],
)


def kernel(x, target):
    tgt = target.astype(jnp.int32)
    tc = _tc_pass(tgt.reshape(N_TOK, 1), x)
    res = _sc_pass(x, tgt)
    a = tc[0, 0] + jnp.sum(res[:, 0, :])
    g = tc[0, 1] + jnp.sum(res[:, 1, :])
    b = tc[0, 2] + jnp.sum(res[:, 2, :])
    k = tc[0, 3] + jnp.sum(res[:, 3, :])
    return k * C0 - EPS * a + EPS * b + (EPS - CONF) * g


# speedup vs baseline: 1.1404x; 1.0180x over previous
import math

import jax
import jax.numpy as jnp
from jax import lax
from jax.experimental import pallas as pl
from jax.experimental.pallas import tpu as pltpu
from jax.experimental.pallas import tpu_sc as plsc

N_TOK = 4096
N_CLS = 32000
PAD = 0
SMOOTHING = 0.1
CONF = 1.0 - SMOOTHING
EPS = SMOOTHING / (N_CLS - 2)
C0 = (N_CLS - 2) * EPS * math.log(EPS) + CONF * math.log(CONF)

TC_ROWS = 2304
SC_ROWS = N_TOK - TC_ROWS

ROW_BLK = 128


HCLS = N_CLS // 2


def _tc_body(tgt_ref, xl_ref, xr_ref, acc_ref):
    @pl.when(pl.program_id(0) == 0)
    def _():
        for q in range(4):
            acc_ref[0, q] = 0.0

    tgt = tgt_ref[...]
    m = tgt != PAD
    mf = m.astype(jnp.float32)
    col = lax.broadcasted_iota(jnp.int32, (ROW_BLK, HCLS), 1)
    tgtm = jnp.where(m, tgt, -1)
    xl = xl_ref[...]
    xr = xr_ref[...]
    acc_ref[0, 0] += jnp.sum((xl + xr) * mf)
    acc_ref[0, 1] += (jnp.sum(jnp.where(col == tgtm, xl, 0.0))
                      + jnp.sum(jnp.where(col + HCLS == tgtm, xr, 0.0)))
    acc_ref[0, 2] += jnp.sum(xl[:, 0:1] * mf)
    acc_ref[0, 3] += jnp.sum(mf)


_tc_pass = pl.pallas_call(
    _tc_body,
    grid=(TC_ROWS // ROW_BLK,),
    in_specs=[
        pl.BlockSpec((ROW_BLK, 1), lambda i: (i, 0)),
        pl.BlockSpec((ROW_BLK, HCLS), lambda i: (i, 0)),
        pl.BlockSpec((ROW_BLK, HCLS), lambda i: (i, 1)),
    ],
    out_specs=pl.BlockSpec((1, 4), lambda i: (0, 0), memory_space=pltpu.SMEM),
    out_shape=jax.ShapeDtypeStruct((1, 4), jnp.float32),
)

L = 16
NC, NS = 2, 16
NW = NC * NS
R_W = SC_ROWS // NW
GR = 8
N_GRP = R_W // GR
CW = 6400
N_CHK = N_CLS // CW
N_TOT = N_GRP * N_CHK


def _sc_body(x_hbm, tgt_hbm, out_hbm, dtgt_v, buf0, buf1, res_v,
             sem0, sem1):
    wid = lax.axis_index("s") * NC + lax.axis_index("c")
    lane_ids = lax.iota(jnp.int32, L)
    row0 = TC_ROWS + wid * R_W
    pltpu.sync_copy(tgt_hbm.at[pl.ds(row0, R_W)], dtgt_v.at[pl.ds(0, R_W)])

    bufs = (buf0, buf1)
    sems = (sem0, sem1)
    zv = jnp.zeros((L,), jnp.float32)
    lane0 = jnp.where(lane_ids == 0, 1.0, 0.0).astype(jnp.float32)

    def _dma(n, s):
        g, c = divmod(n, N_CHK)
        return pltpu.async_copy(
            x_hbm.at[pl.ds(row0 + g * GR, GR), pl.ds(c * CW, CW)],
            bufs[s], sems[s])

    t, mf = [], []
    for g in range(N_GRP):
        tv = dtgt_v[pl.ds(g * GR, L)]
        t.append([tv[r] for r in range(GR)])
        mf.append([jnp.full((L,), jnp.where(tv[r] != PAD, 1.0, 0.0),
                            jnp.float32) for r in range(GR)])

    accd = zv
    accg = zv
    accb = zv
    acck = zv
    cps = [_dma(0, 0), _dma(1, 1)]
    for n in range(N_TOT):
        s = n & 1
        g, c = divmod(n, N_CHK)
        cps[s].wait()
        buf = bufs[s]

        def _chunk(k, ad):
            colb = pl.multiple_of(k * L, L)
            for r in range(GR):
                ad = ad + buf[r, pl.ds(colb, L)] * mf[g][r]
            return ad

        accd = lax.fori_loop(0, CW // L, _chunk, accd)

        for r in range(GR):
            off = t[g][r] - c * CW
            safe = (off >= 0) & (off < CW)
            offc = jnp.where(safe, off, 0)
            lanebit = lax.rem(offc, L)
            start = pl.multiple_of(offc - lanebit, L)
            v = buf[r, pl.ds(start, L)]
            w = jnp.full((L,), jnp.where(safe, 1.0, 0.0), jnp.float32) * mf[g][r]
            lanev = jnp.full((L,), lanebit, jnp.int32)
            accg = accg + v * jnp.where(lane_ids == lanev, w, zv)
        if c == 0:
            for r in range(GR):
                accb = accb + buf[r, pl.ds(0, L)] * (lane0 * mf[g][r])
                acck = acck + lane0 * mf[g][r]
        if n + 2 < N_TOT:
            cps[s] = _dma(n + 2, s)

    res_v[0, :] = accd
    res_v[1, :] = accg
    res_v[2, :] = accb
    res_v[3, :] = acck
    pltpu.sync_copy(res_v, out_hbm.at[wid])


_sc_pass = pl.kernel(
    _sc_body,
    out_type=jax.ShapeDtypeStruct((NW, 4, L), jnp.float32),
    mesh=plsc.VectorSubcoreMesh(core_axis_name="c", subcore_axis_name="s"),
    scratch_types=[
        pltpu.VMEM((R_W + L,), jnp.int32),
        pltpu.VMEM((GR, CW), jnp.float32),
        pltpu.VMEM((GR, CW), jnp.float32),
        pltpu.VMEM((4, L), jnp.float32),
        pltpu.SemaphoreType.DMA,
        pltpu.SemaphoreType.DMA,
    ],
)


def kernel(x, target):
    tgt = target.astype(jnp.int32)
    tc = _tc_pass(tgt.reshape(N_TOK, 1), x, x)
    res = _sc_pass(x, tgt)
    a = tc[0, 0] + jnp.sum(res[:, 0, :])
    g = tc[0, 1] + jnp.sum(res[:, 1, :])
    b = tc[0, 2] + jnp.sum(res[:, 2, :])
    k = tc[0, 3] + jnp.sum(res[:, 3, :])
    return k * C0 - EPS * a + EPS * b + (EPS - CONF) * g
